# Initial kernel scaffold; baseline (speedup 1.0000x reference)
import jax
import jax.numpy as jnp
from jax.experimental import pallas as pl


def kernel(clause_var_idx, clause_sign, clause_ids, params):
    raise NotImplementedError("write your pallas kernel here")


# speedup vs baseline: 2.3525x; 2.3525x over previous
import functools

import jax
import jax.numpy as jnp
from jax import lax
from jax.experimental import pallas as pl
from jax.experimental.pallas import tpu as pltpu
from jax.experimental.pallas import tpu_sc as plsc

V = 50000
C = 200000
L = 600000
FM = 128
ROUNDS = 4

NW = 32
CH = 128
ST = 256
BND = 96
LP = ((L + CH - 1) // CH) * CH
C_PAD = C + 16
VP2 = 2 * V + 16


def _bcast(b, n):
    return jnp.broadcast_to(b, (n,))


def _tc_mlp3(x, layers):
    n = x.shape[0]
    br = 400
    grid = n // br
    (w1, b1), (w2, b2), (w3, b3) = layers
    bst = jnp.zeros((8, FM), jnp.float32)
    bst = bst.at[0].set(b1).at[1].set(b2).at[2].set(b3)

    def body(x_ref, w1r, w2r, w3r, br_, o_ref):
        h = x_ref[...] @ w1r[...] + br_[0:1, :]
        h = jnp.where(h >= 0, h, 0.2 * h)
        h = h @ w2r[...] + br_[1:2, :]
        h = jnp.where(h >= 0, h, 0.2 * h)
        o_ref[...] = h @ w3r[...] + br_[2:3, :]

    wspec = pl.BlockSpec((FM, FM), lambda i: (0, 0))
    return pl.pallas_call(
        body,
        grid=(grid,),
        in_specs=[pl.BlockSpec((br, FM), lambda i: (i, 0)),
                  wspec, wspec, wspec,
                  pl.BlockSpec((8, FM), lambda i: (0, 0))],
        out_specs=pl.BlockSpec((br, FM), lambda i: (i, 0)),
        out_shape=jax.ShapeDtypeStruct((n, FM), jnp.float32),
    )(x, w1, w2, w3, bst)


def _tc_loss_mlps(closs, pos_layers, neg_layers):
    br = 1600
    grid = C // br
    (pw1, pb1), (pw2, pb2), (pw3, pb3) = pos_layers
    (nw1, nb1), (nw2, nb2), (nw3, nb3) = neg_layers
    bst = jnp.zeros((8, FM), jnp.float32)
    bst = bst.at[0].set(pb1).at[1].set(pb2).at[2].set(nb1).at[3].set(nb2)
    bst = bst.at[4].set(jnp.concatenate([pb3, nb3]))

    def body(x_ref, pw1r, pw2r, pw3r, nw1r, nw2r, nw3r, br_, o_ref):
        x = x_ref[...]
        h = x @ pw1r[...] + br_[0:1, :]
        h = jnp.where(h >= 0, h, 0.2 * h)
        h = h @ pw2r[...] + br_[1:2, :]
        h = jnp.where(h >= 0, h, 0.2 * h)
        hp = h @ pw3r[...]
        g = x @ nw1r[...] + br_[2:3, :]
        g = jnp.where(g >= 0, g, 0.2 * g)
        g = g @ nw2r[...] + br_[3:4, :]
        g = jnp.where(g >= 0, g, 0.2 * g)
        hn = g @ nw3r[...]
        o_ref[...] = jnp.concatenate([hp, hn], axis=1) + br_[4:5, :]

    wspec = pl.BlockSpec((FM, FM), lambda i: (0, 0))
    hspec = pl.BlockSpec((FM, FM // 2), lambda i: (0, 0))
    return pl.pallas_call(
        body,
        grid=(grid,),
        in_specs=[pl.BlockSpec((br, FM), lambda i: (i, 0)),
                  wspec, wspec, hspec, wspec, wspec, hspec,
                  pl.BlockSpec((8, FM), lambda i: (0, 0))],
        out_specs=pl.BlockSpec((br, FM), lambda i: (i, 0)),
        out_shape=jax.ShapeDtypeStruct((C, FM), jnp.float32),
    )(closs, pw1, pw2, pw3, nw1, nw2, nw3, bst)


def _tc_gates(variables, msg, fg, ug, og, gamma, beta):
    br = 400
    grid = V // br
    (fw1, fb1), (fw2, fb2), (fw3, fb3) = fg
    (uw1, ub1), (uw2, ub2), (uw3, ub3) = ug
    (ow1, ob1), (ow2, ob2), (ow3, ob3) = og
    bst = jnp.zeros((8, FM), jnp.float32)
    bst = (bst.at[0].set(fb1).at[1].set(fb2).at[2].set(fb3)
              .at[3].set(ub1).at[4].set(ub2).at[5].set(ub3)
              .at[6].set(ob1).at[7].set(ob2))
    bst2 = jnp.zeros((8, FM), jnp.float32)
    bst2 = (bst2.at[0].set(gamma).at[1].set(beta)
                .at[2].set(ow3[:, 0]).at[3].set(_bcast(ob3[0], FM)))

    def body(v_ref, m_ref, fw1v, fw1m, fw2r, fw3r,
             uw1v, uw1m, uw2r, uw3r, ow1r, ow2r, br_, br2_, nv_ref, lg_ref):
        v = v_ref[...]
        m = m_ref[...]
        h = v @ fw1v[...] + m @ fw1m[...] + br_[0:1, :]
        h = jnp.where(h >= 0, h, 0.2 * h)
        h = h @ fw2r[...] + br_[1:2, :]
        h = jnp.where(h >= 0, h, 0.2 * h)
        f = jax.nn.sigmoid(h @ fw3r[...] + br_[2:3, :])
        h = v @ uw1v[...] + m @ uw1m[...] + br_[3:4, :]
        h = jnp.where(h >= 0, h, 0.2 * h)
        h = h @ uw2r[...] + br_[4:5, :]
        h = jnp.where(h >= 0, h, 0.2 * h)
        u = h @ uw3r[...] + br_[5:6, :]
        nv = (1.0 - f) * v + f * u
        mu = jnp.mean(nv, axis=1, keepdims=True)
        d = nv - mu
        var = jnp.mean(d * d, axis=1, keepdims=True)
        ln = d * lax.rsqrt(var + 1e-3) * br2_[0:1, :] + br2_[1:2, :]
        nv_ref[...] = ln
        h = ln @ ow1r[...] + br_[6:7, :]
        h = jnp.where(h >= 0, h, 0.2 * h)
        h = h @ ow2r[...] + br_[7:8, :]
        h = jnp.where(h >= 0, h, 0.2 * h)
        lg_ref[...] = (jnp.sum(h * br2_[2:3, :], axis=1, keepdims=True)
                       + br2_[3:4, 0:1])

    wspec = pl.BlockSpec((FM, FM), lambda i: (0, 0))
    bspec = pl.BlockSpec((8, FM), lambda i: (0, 0))
    return pl.pallas_call(
        body,
        grid=(grid,),
        in_specs=[pl.BlockSpec((br, FM), lambda i: (i, 0)),
                  pl.BlockSpec((br, FM), lambda i: (i, 0)),
                  wspec, wspec, wspec, wspec,
                  wspec, wspec, wspec, wspec,
                  wspec, wspec, bspec, bspec],
        out_specs=[pl.BlockSpec((br, FM), lambda i: (i, 0)),
                   pl.BlockSpec((br, 1), lambda i: (i, 0))],
        out_shape=[jax.ShapeDtypeStruct((V, FM), jnp.float32),
                   jax.ShapeDtypeStruct((V, 1), jnp.float32)],
    )(variables, msg, fw1[:FM], fw1[FM:], fw2, fw3,
      uw1[:FM], uw1[FM:], uw2, uw3, ow1, ow2, bst, bst2)


def _sc_seg_reduce(is_prod, ncol, nrow, table, gidx, segid, bnd):
    nk = ncol // 16
    ident = 1.0 if is_prod else 0.0
    mesh = plsc.VectorSubcoreMesh(core_axis_name="c", subcore_axis_name="s")

    @functools.partial(
        pl.kernel,
        out_type=jax.ShapeDtypeStruct((nrow, ncol), jnp.float32),
        mesh=mesh,
        compiler_params=pltpu.CompilerParams(use_tc_tiling_on_sc=False),
        scratch_types=[
            pltpu.VMEM((CH,), jnp.int32),
            pltpu.VMEM((CH + 16,), jnp.int32),
            pltpu.VMEM((BND,), jnp.int32),
            pltpu.VMEM((CH, ncol), jnp.float32),
            pltpu.VMEM((ST, ncol), jnp.float32),
            pltpu.VMEM((ST,), jnp.int32),
            pltpu.VMEM((ncol,), jnp.float32),
            pltpu.SMEM((8,), jnp.int32),
        ],
    )
    def k(table_h, gidx_h, seg_h, bnd_h, out_h,
          idxb, cidb, bnds, rows, stage, sidx, accv, sm):
        w = lax.axis_index("c") * 16 + lax.axis_index("s")
        pltpu.sync_copy(bnd_h, bnds)
        lo = bnds[pl.ds(w, 16)][0]
        hi = bnds[pl.ds(w + 1, 16)][0]
        rlo = bnds[pl.ds(w + 48, 16)][0]
        rhi = bnds[pl.ds(w + 49, 16)][0]

        ident_v = jnp.full((16,), ident, jnp.float32)
        dump_v = jnp.full((16,), nrow - 1, jnp.int32)
        lane0 = lax.iota(jnp.int32, 16) == 0

        @pl.loop(0, ST)
        def _(r):
            for k2 in range(nk):
                stage[r, pl.ds(16 * k2, 16)] = ident_v

        for k2 in range(ST // 16):
            sidx[pl.ds(16 * k2, 16)] = dump_v
        for k2 in range(nk):
            accv[pl.ds(16 * k2, 16)] = ident_v
        sm[0] = jnp.int32(-1)
        sm[1] = jnp.int32(0)

        def fill_body(i, rr):
            pltpu.sync_copy(stage, out_h.at[pl.ds(rr, ST)])
            return rr + ST

        rr = lax.fori_loop(0, (rhi - rlo) // ST, fill_body, rlo)
        for s in (128, 64, 32, 16, 8, 4, 2, 1):
            do = rr + s <= rhi

            @pl.when(do)
            def _(s=s, rr=rr):
                pltpu.sync_copy(stage.at[pl.ds(0, s)], out_h.at[pl.ds(rr, s)])

            rr = jnp.where(do, rr + s, rr)

        def write_seg():
            prev = sm[0]
            nst = sm[1]
            for k2 in range(nk):
                a = accv[pl.ds(16 * k2, 16)]
                stage[nst, pl.ds(16 * k2, 16)] = (1.0 / a) if is_prod else a
                accv[pl.ds(16 * k2, 16)] = ident_v
            prev_v = jnp.full((16,), prev, jnp.int32)
            sidx[pl.ds(nst, 16)] = jnp.where(lane0, prev_v, dump_v)
            sm[1] = nst + 1

        def lit_body(j, _, base):
            g = base + j
            valid = jnp.logical_and(g >= lo, g < hi)

            @pl.when(valid)
            def _():
                cid = cidb[pl.ds(j, 16)][0]
                prev = sm[0]

                @pl.when(jnp.logical_and(cid != prev, prev >= 0))
                def _():
                    write_seg()

                for k2 in range(nk):
                    x = rows[j, pl.ds(16 * k2, 16)]
                    a = accv[pl.ds(16 * k2, 16)]
                    if is_prod:
                        a = a * (1.0 + jnp.exp(x))
                    else:
                        a = a + x
                    accv[pl.ds(16 * k2, 16)] = a
                sm[0] = cid

            return 0

        def chunk_body(ci, _):
            base = ci * CH
            pltpu.sync_copy(gidx_h.at[pl.ds(base, CH)], idxb)
            pltpu.sync_copy(seg_h.at[pl.ds(base, CH)], cidb.at[pl.ds(0, CH)])
            pltpu.sync_copy(table_h.at[idxb], rows)
            lax.fori_loop(0, CH, lambda j, c: lit_body(j, c, base), 0)

            @pl.when(sm[1] >= ST - CH - 16)
            def _():
                pltpu.sync_copy(stage, out_h.at[sidx])
                sm[1] = jnp.int32(0)

            return 0

        lax.fori_loop(lo // CH, (hi + CH - 1) // CH, chunk_body, 0)

        @pl.when(sm[0] >= 0)
        def _():
            write_seg()

        pltpu.sync_copy(stage, out_h.at[sidx])

    return k(table, gidx, segid, bnd)


def _worker_bounds(seg_sorted, nrow):
    pos = jnp.arange(NW, dtype=jnp.int32) * (L // NW)
    heads = seg_sorted[pos]
    b = jnp.searchsorted(seg_sorted, heads, side="left").astype(jnp.int32)
    blit = jnp.concatenate([b, jnp.array([L], jnp.int32)])
    capped = jnp.minimum(blit[:NW], L - 1)
    brow_head = jnp.where(blit[:NW] < L, seg_sorted[capped],
                          jnp.int32(nrow))
    brow = jnp.concatenate([brow_head, jnp.array([nrow], jnp.int32)])
    brow = brow.at[0].set(0)
    return jnp.concatenate([jnp.pad(blit, (0, 15)),
                            jnp.pad(brow, (0, BND - 48 - 33))])


def kernel(clause_var_idx, clause_sign, clause_ids, params):
    sign = clause_sign.astype(jnp.int32)
    inv = 1 - sign
    gidx2 = clause_var_idx + V * inv
    dst4 = 2 * clause_var_idx + inv
    src4 = 2 * clause_ids + inv
    sd, ss = lax.sort([dst4, src4], num_keys=1)

    bnd2 = _worker_bounds(clause_ids, C_PAD)
    bnd4 = _worker_bounds(sd, VP2)

    pad = LP - L
    gidx2p = jnp.pad(gidx2, (0, pad))
    cidsp = jnp.pad(clause_ids, (0, pad))
    ssp = jnp.pad(ss, (0, pad))
    sdp = jnp.pad(sd, (0, pad))

    variables = 0.25 * jax.random.truncated_normal(
        jax.random.key(1), -2.0, 2.0, (V, FM), dtype=jnp.float32)

    step_logits = []
    for _ in range(ROUNDS):
        q = _tc_mlp3(variables, params["variables_query"])
        qext = jnp.concatenate([q, -q], axis=0)
        closs_full = _sc_seg_reduce(True, FM, C_PAD, qext, gidx2p, cidsp,
                                    bnd2)
        lc = _tc_loss_mlps(closs_full[:C], params["query_pos_inter"],
                           params["query_neg_inter"])
        msg_full = _sc_seg_reduce(False, FM // 2, VP2,
                                  lc.reshape(2 * C, FM // 2), ssp, sdp, bnd4)
        msg = msg_full[:2 * V].reshape(V, FM)
        variables, logit = _tc_gates(variables, msg, params["forget_gate"],
                                     params["update_gate"],
                                     params["variables_output"],
                                     params["ln_gamma"], params["ln_beta"])
        step_logits.append(logit)
    return jnp.stack(step_logits, axis=0)


# speedup vs baseline: 5.8637x; 2.4926x over previous
import functools

import jax
import jax.numpy as jnp
from jax import lax
from jax.experimental import pallas as pl
from jax.experimental.pallas import tpu as pltpu
from jax.experimental.pallas import tpu_sc as plsc

V = 50000
C = 200000
L = 600000
FM = 128
ROUNDS = 4

NW = 32
CH = 128
ST = 256
BND = 96
LP = ((L + CH - 1) // CH) * CH
C_PAD = C + 16
VP2 = 2 * V + 16


def _bcast(b, n):
    return jnp.broadcast_to(b, (n,))


def _tc_mlp3(x, layers):
    n = x.shape[0]
    br = 400
    grid = n // br
    (w1, b1), (w2, b2), (w3, b3) = layers
    bst = jnp.zeros((8, FM), jnp.float32)
    bst = bst.at[0].set(b1).at[1].set(b2).at[2].set(b3)

    def body(x_ref, w1r, w2r, w3r, br_, o_ref):
        h = x_ref[...] @ w1r[...] + br_[0:1, :]
        h = jnp.where(h >= 0, h, 0.2 * h)
        h = h @ w2r[...] + br_[1:2, :]
        h = jnp.where(h >= 0, h, 0.2 * h)
        o_ref[...] = h @ w3r[...] + br_[2:3, :]

    wspec = pl.BlockSpec((FM, FM), lambda i: (0, 0))
    return pl.pallas_call(
        body,
        grid=(grid,),
        in_specs=[pl.BlockSpec((br, FM), lambda i: (i, 0)),
                  wspec, wspec, wspec,
                  pl.BlockSpec((8, FM), lambda i: (0, 0))],
        out_specs=pl.BlockSpec((br, FM), lambda i: (i, 0)),
        out_shape=jax.ShapeDtypeStruct((n, FM), jnp.float32),
    )(x, w1, w2, w3, bst)


def _tc_loss_mlps(closs, pos_layers, neg_layers):
    br = 1600
    grid = C // br
    (pw1, pb1), (pw2, pb2), (pw3, pb3) = pos_layers
    (nw1, nb1), (nw2, nb2), (nw3, nb3) = neg_layers
    bst = jnp.zeros((8, FM), jnp.float32)
    bst = bst.at[0].set(pb1).at[1].set(pb2).at[2].set(nb1).at[3].set(nb2)
    bst = bst.at[4].set(jnp.concatenate([pb3, nb3]))

    def body(x_ref, pw1r, pw2r, pw3r, nw1r, nw2r, nw3r, br_, o_ref):
        x = x_ref[...]
        h = x @ pw1r[...] + br_[0:1, :]
        h = jnp.where(h >= 0, h, 0.2 * h)
        h = h @ pw2r[...] + br_[1:2, :]
        h = jnp.where(h >= 0, h, 0.2 * h)
        hp = h @ pw3r[...]
        g = x @ nw1r[...] + br_[2:3, :]
        g = jnp.where(g >= 0, g, 0.2 * g)
        g = g @ nw2r[...] + br_[3:4, :]
        g = jnp.where(g >= 0, g, 0.2 * g)
        hn = g @ nw3r[...]
        o_ref[...] = jnp.concatenate([hp, hn], axis=1) + br_[4:5, :]

    wspec = pl.BlockSpec((FM, FM), lambda i: (0, 0))
    hspec = pl.BlockSpec((FM, FM // 2), lambda i: (0, 0))
    return pl.pallas_call(
        body,
        grid=(grid,),
        in_specs=[pl.BlockSpec((br, FM), lambda i: (i, 0)),
                  wspec, wspec, hspec, wspec, wspec, hspec,
                  pl.BlockSpec((8, FM), lambda i: (0, 0))],
        out_specs=pl.BlockSpec((br, FM), lambda i: (i, 0)),
        out_shape=jax.ShapeDtypeStruct((C, FM), jnp.float32),
    )(closs, pw1, pw2, pw3, nw1, nw2, nw3, bst)


def _tc_gates(variables, msg, fg, ug, og, gamma, beta):
    br = 400
    grid = V // br
    (fw1, fb1), (fw2, fb2), (fw3, fb3) = fg
    (uw1, ub1), (uw2, ub2), (uw3, ub3) = ug
    (ow1, ob1), (ow2, ob2), (ow3, ob3) = og
    bst = jnp.zeros((8, FM), jnp.float32)
    bst = (bst.at[0].set(fb1).at[1].set(fb2).at[2].set(fb3)
              .at[3].set(ub1).at[4].set(ub2).at[5].set(ub3)
              .at[6].set(ob1).at[7].set(ob2))
    bst2 = jnp.zeros((8, FM), jnp.float32)
    bst2 = (bst2.at[0].set(gamma).at[1].set(beta)
                .at[2].set(ow3[:, 0]).at[3].set(_bcast(ob3[0], FM)))

    def body(v_ref, m_ref, fw1v, fw1m, fw2r, fw3r,
             uw1v, uw1m, uw2r, uw3r, ow1r, ow2r, br_, br2_, nv_ref, lg_ref):
        v = v_ref[...]
        m = m_ref[...]
        h = v @ fw1v[...] + m @ fw1m[...] + br_[0:1, :]
        h = jnp.where(h >= 0, h, 0.2 * h)
        h = h @ fw2r[...] + br_[1:2, :]
        h = jnp.where(h >= 0, h, 0.2 * h)
        f = jax.nn.sigmoid(h @ fw3r[...] + br_[2:3, :])
        h = v @ uw1v[...] + m @ uw1m[...] + br_[3:4, :]
        h = jnp.where(h >= 0, h, 0.2 * h)
        h = h @ uw2r[...] + br_[4:5, :]
        h = jnp.where(h >= 0, h, 0.2 * h)
        u = h @ uw3r[...] + br_[5:6, :]
        nv = (1.0 - f) * v + f * u
        mu = jnp.mean(nv, axis=1, keepdims=True)
        d = nv - mu
        var = jnp.mean(d * d, axis=1, keepdims=True)
        ln = d * lax.rsqrt(var + 1e-3) * br2_[0:1, :] + br2_[1:2, :]
        nv_ref[...] = ln
        h = ln @ ow1r[...] + br_[6:7, :]
        h = jnp.where(h >= 0, h, 0.2 * h)
        h = h @ ow2r[...] + br_[7:8, :]
        h = jnp.where(h >= 0, h, 0.2 * h)
        lg_ref[...] = (jnp.sum(h * br2_[2:3, :], axis=1, keepdims=True)
                       + br2_[3:4, 0:1])

    wspec = pl.BlockSpec((FM, FM), lambda i: (0, 0))
    bspec = pl.BlockSpec((8, FM), lambda i: (0, 0))
    return pl.pallas_call(
        body,
        grid=(grid,),
        in_specs=[pl.BlockSpec((br, FM), lambda i: (i, 0)),
                  pl.BlockSpec((br, FM), lambda i: (i, 0)),
                  wspec, wspec, wspec, wspec,
                  wspec, wspec, wspec, wspec,
                  wspec, wspec, bspec, bspec],
        out_specs=[pl.BlockSpec((br, FM), lambda i: (i, 0)),
                   pl.BlockSpec((br, 1), lambda i: (i, 0))],
        out_shape=[jax.ShapeDtypeStruct((V, FM), jnp.float32),
                   jax.ShapeDtypeStruct((V, 1), jnp.float32)],
    )(variables, msg, fw1[:FM], fw1[FM:], fw2, fw3,
      uw1[:FM], uw1[FM:], uw2, uw3, ow1, ow2, bst, bst2)


def _sc_seg_reduce(is_prod, ncol, nrow, table, gidx, segid, bnd):
    nk = ncol // 16
    ident = 1.0 if is_prod else 0.0
    mesh = plsc.VectorSubcoreMesh(core_axis_name="c", subcore_axis_name="s")

    @functools.partial(
        pl.kernel,
        out_type=jax.ShapeDtypeStruct((nrow, ncol), jnp.float32),
        mesh=mesh,
        compiler_params=pltpu.CompilerParams(use_tc_tiling_on_sc=False),
        scratch_types=[
            pltpu.VMEM((CH,), jnp.int32),
            pltpu.VMEM((CH + 16,), jnp.int32),
            pltpu.VMEM((BND,), jnp.int32),
            pltpu.VMEM((CH, ncol), jnp.float32),
            pltpu.VMEM((ST, ncol), jnp.float32),
            pltpu.VMEM((ST,), jnp.int32),
            pltpu.SMEM((8,), jnp.int32),
        ],
    )
    def k(table_h, gidx_h, seg_h, bnd_h, out_h,
          idxb, cidb, bnds, rows, stage, sidx, sm):
        w = lax.axis_index("c") * 16 + lax.axis_index("s")
        pltpu.sync_copy(bnd_h, bnds)
        lo = bnds[pl.ds(w, 16)][0]
        hi = bnds[pl.ds(w + 1, 16)][0]
        rlo = bnds[pl.ds(w + 48, 16)][0]
        rhi = bnds[pl.ds(w + 49, 16)][0]

        ident_v = jnp.full((16,), ident, jnp.float32)
        dump_v = jnp.full((16,), nrow - 1, jnp.int32)

        @pl.loop(0, ST)
        def _(r):
            for k2 in range(nk):
                stage[r, pl.ds(16 * k2, 16)] = ident_v

        for k2 in range(ST // 16):
            sidx[pl.ds(16 * k2, 16)] = dump_v
        sm[1] = jnp.int32(0)

        def fill_body(i, rr):
            pltpu.sync_copy(stage, out_h.at[pl.ds(rr, ST)])
            return rr + ST

        rr = lax.fori_loop(0, (rhi - rlo) // ST, fill_body, rlo)
        for s in (128, 64, 32, 16, 8, 4, 2, 1):
            do = rr + s <= rhi

            @pl.when(do)
            def _(s=s, rr=rr):
                pltpu.sync_copy(stage.at[pl.ds(0, s)], out_h.at[pl.ds(rr, s)])

            rr = jnp.where(do, rr + s, rr)

        sm[0] = jnp.int32(-1)

        def finalize_and_scatter():
            if is_prod:
                @pl.loop(0, sm[1])
                def _(r):
                    for k2 in range(nk):
                        stage[r, pl.ds(16 * k2, 16)] = (
                            1.0 / stage[r, pl.ds(16 * k2, 16)])
            pltpu.sync_copy(stage, out_h.at[sidx])

        def lit_step(jl, guarded, base):
            cid = cidb[pl.ds(jl, 16)][0]

            def boundary():
                @pl.when(sm[1] >= ST - 16)
                def _():
                    finalize_and_scatter()
                    sm[1] = jnp.int32(0)

                s = sm[1]
                for k2 in range(nk):
                    stage[s, pl.ds(16 * k2, 16)] = ident_v
                cid_v = jnp.full((16,), cid, jnp.int32)
                l0 = lax.iota(jnp.int32, 16) == 0
                sidx[pl.ds(s, 16)] = jnp.where(l0, cid_v, dump_v)
                sm[1] = s + 1
                sm[0] = cid

            def work():
                @pl.when(cid != sm[0])
                def _():
                    boundary()

                s1 = sm[1] - 1
                for k2 in range(nk):
                    x = rows[jl, pl.ds(16 * k2, 16)]
                    a = stage[s1, pl.ds(16 * k2, 16)]
                    if is_prod:
                        a = a * (1.0 + jnp.exp(x))
                    else:
                        a = a + x
                    stage[s1, pl.ds(16 * k2, 16)] = a

            if guarded:
                g = base + jl

                @pl.when(jnp.logical_and(g >= lo, g < hi))
                def _():
                    work()
            else:
                work()

        UNR = 8

        def make_chunk_body(guarded):
            def chunk_body(ci, car):
                base = ci * CH
                pltpu.sync_copy(gidx_h.at[pl.ds(base, CH)], idxb)
                pltpu.sync_copy(seg_h.at[pl.ds(base, CH)],
                                cidb.at[pl.ds(0, CH)])
                pltpu.sync_copy(table_h.at[idxb], rows)

                def blk(b, c):
                    for u in range(UNR):
                        lit_step(b * UNR + u, guarded, base)
                    return c

                return lax.fori_loop(0, CH // UNR, blk, car)

            return chunk_body

        c0 = lo // CH
        c1 = (hi + CH - 1) // CH
        cm0 = (lo + CH - 1) // CH
        cm1 = hi // CH
        lax.fori_loop(c0, jnp.minimum(cm0, c1), make_chunk_body(True), 0)
        lax.fori_loop(cm0, jnp.maximum(cm0, cm1), make_chunk_body(False), 0)
        lax.fori_loop(jnp.maximum(cm1, jnp.minimum(cm0, c1)), c1,
                      make_chunk_body(True), 0)

        finalize_and_scatter()

    return k(table, gidx, segid, bnd)


def _worker_bounds(seg_sorted, nrow):
    pos = jnp.arange(NW, dtype=jnp.int32) * (L // NW)
    heads = seg_sorted[pos]
    b = jnp.searchsorted(seg_sorted, heads, side="left").astype(jnp.int32)
    blit = jnp.concatenate([b, jnp.array([L], jnp.int32)])
    capped = jnp.minimum(blit[:NW], L - 1)
    brow_head = jnp.where(blit[:NW] < L, seg_sorted[capped],
                          jnp.int32(nrow))
    brow = jnp.concatenate([brow_head, jnp.array([nrow], jnp.int32)])
    brow = brow.at[0].set(0)
    return jnp.concatenate([jnp.pad(blit, (0, 15)),
                            jnp.pad(brow, (0, BND - 48 - 33))])


def kernel(clause_var_idx, clause_sign, clause_ids, params):
    sign = clause_sign.astype(jnp.int32)
    inv = 1 - sign
    gidx2 = clause_var_idx + V * inv
    dst4 = 2 * clause_var_idx + inv
    src4 = 2 * clause_ids + inv
    sd, ss = lax.sort([dst4, src4], num_keys=1)

    bnd2 = _worker_bounds(clause_ids, C_PAD)
    bnd4 = _worker_bounds(sd, VP2)

    pad = LP - L
    gidx2p = jnp.pad(gidx2, (0, pad))
    cidsp = jnp.pad(clause_ids, (0, pad))
    ssp = jnp.pad(ss, (0, pad))
    sdp = jnp.pad(sd, (0, pad))

    variables = 0.25 * jax.random.truncated_normal(
        jax.random.key(1), -2.0, 2.0, (V, FM), dtype=jnp.float32)

    step_logits = []
    for _ in range(ROUNDS):
        q = _tc_mlp3(variables, params["variables_query"])
        qext = jnp.concatenate([q, -q], axis=0)
        closs_full = _sc_seg_reduce(True, FM, C_PAD, qext, gidx2p, cidsp,
                                    bnd2)
        lc = _tc_loss_mlps(closs_full[:C], params["query_pos_inter"],
                           params["query_neg_inter"])
        msg_full = _sc_seg_reduce(False, FM // 2, VP2,
                                  lc.reshape(2 * C, FM // 2), ssp, sdp, bnd4)
        msg = msg_full[:2 * V].reshape(V, FM)
        variables, logit = _tc_gates(variables, msg, params["forget_gate"],
                                     params["update_gate"],
                                     params["variables_output"],
                                     params["ln_gamma"], params["ln_beta"])
        step_logits.append(logit)
    return jnp.stack(step_logits, axis=0)


# speedup vs baseline: 6.1630x; 1.0510x over previous
import functools

import jax
import jax.numpy as jnp
from jax import lax
from jax.experimental import pallas as pl
from jax.experimental.pallas import tpu as pltpu
from jax.experimental.pallas import tpu_sc as plsc

V = 50000
C = 200000
L = 600000
FM = 128
ROUNDS = 4

NW = 32
CH = 128
ST = 256
GCH = 1024
BND = 96
LP = ((L + CH - 1) // CH) * CH
C_PAD = C + 16
VP2 = 2 * V + 16


def _bcast(b, n):
    return jnp.broadcast_to(b, (n,))


def _tc_mlp3(x, layers):
    n = x.shape[0]
    br = 400
    grid = n // br
    (w1, b1), (w2, b2), (w3, b3) = layers
    bst = jnp.zeros((8, FM), jnp.float32)
    bst = bst.at[0].set(b1).at[1].set(b2).at[2].set(b3)

    def body(x_ref, w1r, w2r, w3r, br_, o_ref):
        h = x_ref[...] @ w1r[...] + br_[0:1, :]
        h = jnp.where(h >= 0, h, 0.2 * h)
        h = h @ w2r[...] + br_[1:2, :]
        h = jnp.where(h >= 0, h, 0.2 * h)
        o_ref[...] = h @ w3r[...] + br_[2:3, :]

    wspec = pl.BlockSpec((FM, FM), lambda i: (0, 0))
    return pl.pallas_call(
        body,
        grid=(grid,),
        in_specs=[pl.BlockSpec((br, FM), lambda i: (i, 0)),
                  wspec, wspec, wspec,
                  pl.BlockSpec((8, FM), lambda i: (0, 0))],
        out_specs=pl.BlockSpec((br, FM), lambda i: (i, 0)),
        out_shape=jax.ShapeDtypeStruct((n, FM), jnp.float32),
    )(x, w1, w2, w3, bst)


def _tc_loss_mlps(closs, pos_layers, neg_layers):
    br = 1600
    grid = C // br
    (pw1, pb1), (pw2, pb2), (pw3, pb3) = pos_layers
    (nw1, nb1), (nw2, nb2), (nw3, nb3) = neg_layers
    bst = jnp.zeros((8, FM), jnp.float32)
    bst = bst.at[0].set(pb1).at[1].set(pb2).at[2].set(nb1).at[3].set(nb2)
    bst = bst.at[4].set(jnp.concatenate([pb3, nb3]))

    def body(x_ref, pw1r, pw2r, pw3r, nw1r, nw2r, nw3r, br_, o_ref):
        x = x_ref[...]
        h = x @ pw1r[...] + br_[0:1, :]
        h = jnp.where(h >= 0, h, 0.2 * h)
        h = h @ pw2r[...] + br_[1:2, :]
        h = jnp.where(h >= 0, h, 0.2 * h)
        hp = h @ pw3r[...]
        g = x @ nw1r[...] + br_[2:3, :]
        g = jnp.where(g >= 0, g, 0.2 * g)
        g = g @ nw2r[...] + br_[3:4, :]
        g = jnp.where(g >= 0, g, 0.2 * g)
        hn = g @ nw3r[...]
        o_ref[...] = jnp.concatenate([hp, hn], axis=1) + br_[4:5, :]

    wspec = pl.BlockSpec((FM, FM), lambda i: (0, 0))
    hspec = pl.BlockSpec((FM, FM // 2), lambda i: (0, 0))
    return pl.pallas_call(
        body,
        grid=(grid,),
        in_specs=[pl.BlockSpec((br, FM), lambda i: (i, 0)),
                  wspec, wspec, hspec, wspec, wspec, hspec,
                  pl.BlockSpec((8, FM), lambda i: (0, 0))],
        out_specs=pl.BlockSpec((br, FM), lambda i: (i, 0)),
        out_shape=jax.ShapeDtypeStruct((C, FM), jnp.float32),
    )(closs, pw1, pw2, pw3, nw1, nw2, nw3, bst)


def _tc_gates(variables, msg, fg, ug, og, gamma, beta):
    br = 400
    grid = V // br
    (fw1, fb1), (fw2, fb2), (fw3, fb3) = fg
    (uw1, ub1), (uw2, ub2), (uw3, ub3) = ug
    (ow1, ob1), (ow2, ob2), (ow3, ob3) = og
    bst = jnp.zeros((8, FM), jnp.float32)
    bst = (bst.at[0].set(fb1).at[1].set(fb2).at[2].set(fb3)
              .at[3].set(ub1).at[4].set(ub2).at[5].set(ub3)
              .at[6].set(ob1).at[7].set(ob2))
    bst2 = jnp.zeros((8, FM), jnp.float32)
    bst2 = (bst2.at[0].set(gamma).at[1].set(beta)
                .at[2].set(ow3[:, 0]).at[3].set(_bcast(ob3[0], FM)))

    def body(v_ref, m_ref, fw1v, fw1m, fw2r, fw3r,
             uw1v, uw1m, uw2r, uw3r, ow1r, ow2r, br_, br2_, nv_ref, lg_ref):
        v = v_ref[...]
        m = m_ref[...]
        h = v @ fw1v[...] + m @ fw1m[...] + br_[0:1, :]
        h = jnp.where(h >= 0, h, 0.2 * h)
        h = h @ fw2r[...] + br_[1:2, :]
        h = jnp.where(h >= 0, h, 0.2 * h)
        f = jax.nn.sigmoid(h @ fw3r[...] + br_[2:3, :])
        h = v @ uw1v[...] + m @ uw1m[...] + br_[3:4, :]
        h = jnp.where(h >= 0, h, 0.2 * h)
        h = h @ uw2r[...] + br_[4:5, :]
        h = jnp.where(h >= 0, h, 0.2 * h)
        u = h @ uw3r[...] + br_[5:6, :]
        nv = (1.0 - f) * v + f * u
        mu = jnp.mean(nv, axis=1, keepdims=True)
        d = nv - mu
        var = jnp.mean(d * d, axis=1, keepdims=True)
        ln = d * lax.rsqrt(var + 1e-3) * br2_[0:1, :] + br2_[1:2, :]
        nv_ref[...] = ln
        h = ln @ ow1r[...] + br_[6:7, :]
        h = jnp.where(h >= 0, h, 0.2 * h)
        h = h @ ow2r[...] + br_[7:8, :]
        h = jnp.where(h >= 0, h, 0.2 * h)
        lg_ref[...] = (jnp.sum(h * br2_[2:3, :], axis=1, keepdims=True)
                       + br2_[3:4, 0:1])

    wspec = pl.BlockSpec((FM, FM), lambda i: (0, 0))
    bspec = pl.BlockSpec((8, FM), lambda i: (0, 0))
    return pl.pallas_call(
        body,
        grid=(grid,),
        in_specs=[pl.BlockSpec((br, FM), lambda i: (i, 0)),
                  pl.BlockSpec((br, FM), lambda i: (i, 0)),
                  wspec, wspec, wspec, wspec,
                  wspec, wspec, wspec, wspec,
                  wspec, wspec, bspec, bspec],
        out_specs=[pl.BlockSpec((br, FM), lambda i: (i, 0)),
                   pl.BlockSpec((br, 1), lambda i: (i, 0))],
        out_shape=[jax.ShapeDtypeStruct((V, FM), jnp.float32),
                   jax.ShapeDtypeStruct((V, 1), jnp.float32)],
    )(variables, msg, fw1[:FM], fw1[FM:], fw2, fw3,
      uw1[:FM], uw1[FM:], uw2, uw3, ow1, ow2, bst, bst2)


def _sc_seg_reduce(is_prod, ncol, nrow, table, gidx, segid, bnd):
    nk = ncol // 16
    ident = 1.0 if is_prod else 0.0
    gch = GCH // 2 if nk > 4 else GCH
    mesh = plsc.VectorSubcoreMesh(core_axis_name="c", subcore_axis_name="s")

    @functools.partial(
        pl.kernel,
        out_type=jax.ShapeDtypeStruct((nrow, ncol), jnp.float32),
        mesh=mesh,
        compiler_params=pltpu.CompilerParams(use_tc_tiling_on_sc=False),
        scratch_types=[
            pltpu.VMEM((gch,), jnp.int32),
            pltpu.VMEM((gch + 16,), jnp.int32),
            pltpu.VMEM((BND,), jnp.int32),
            pltpu.VMEM((CH, ncol), jnp.float32),
            pltpu.VMEM((CH, ncol), jnp.float32),
            pltpu.VMEM((ST, ncol), jnp.float32),
            pltpu.VMEM((ST,), jnp.int32),
            pltpu.SMEM((8,), jnp.int32),
            pltpu.SemaphoreType.DMA,
            pltpu.SemaphoreType.DMA,
        ],
    )
    def k(table_h, gidx_h, seg_h, bnd_h, out_h,
          idxg, cidg, bnds, rows0, rows1, stage, sidx, sm, sem0, sem1):
        w = lax.axis_index("c") * 16 + lax.axis_index("s")
        pltpu.sync_copy(bnd_h, bnds)
        lo = bnds[pl.ds(w, 16)][0]
        hi = bnds[pl.ds(w + 1, 16)][0]
        rlo = bnds[pl.ds(w + 48, 16)][0]
        rhi = bnds[pl.ds(w + 49, 16)][0]

        ident_v = jnp.full((16,), ident, jnp.float32)
        dump_v = jnp.full((16,), nrow - 1, jnp.int32)

        @pl.loop(0, ST)
        def _(r):
            for k2 in range(nk):
                stage[r, pl.ds(16 * k2, 16)] = ident_v

        for k2 in range(ST // 16):
            sidx[pl.ds(16 * k2, 16)] = dump_v
        sm[1] = jnp.int32(0)

        def fill_body(i, rr):
            pltpu.sync_copy(stage, out_h.at[pl.ds(rr, ST)])
            return rr + ST

        rr = lax.fori_loop(0, (rhi - rlo) // ST, fill_body, rlo)
        for s in (128, 64, 32, 16, 8, 4, 2, 1):
            do = rr + s <= rhi

            @pl.when(do)
            def _(s=s, rr=rr):
                pltpu.sync_copy(stage.at[pl.ds(0, s)], out_h.at[pl.ds(rr, s)])

            rr = jnp.where(do, rr + s, rr)

        sm[0] = jnp.int32(-1)

        def finalize_and_scatter():
            if is_prod:
                @pl.loop(0, sm[1])
                def _(r):
                    for k2 in range(nk):
                        stage[r, pl.ds(16 * k2, 16)] = (
                            1.0 / stage[r, pl.ds(16 * k2, 16)])
            pltpu.sync_copy(stage, out_h.at[sidx])

        def lit_step(jl, jr, base, rows):
            cid = cidg[pl.ds(jl, 16)][0]

            def boundary():
                @pl.when(sm[1] >= ST - 16)
                def _():
                    finalize_and_scatter()
                    sm[1] = jnp.int32(0)

                s = sm[1]
                for k2 in range(nk):
                    stage[s, pl.ds(16 * k2, 16)] = ident_v
                cid_v = jnp.full((16,), cid, jnp.int32)
                l0 = lax.iota(jnp.int32, 16) == 0
                sidx[pl.ds(s, 16)] = jnp.where(l0, cid_v, dump_v)
                sm[1] = s + 1
                sm[0] = cid

            def work():
                @pl.when(cid != sm[0])
                def _():
                    boundary()

                s1 = sm[1] - 1
                for k2 in range(nk):
                    x = rows[jr, pl.ds(16 * k2, 16)]
                    a = stage[s1, pl.ds(16 * k2, 16)]
                    if is_prod:
                        a = a * (1.0 + jnp.exp(x))
                    else:
                        a = a + x
                    stage[s1, pl.ds(16 * k2, 16)] = a

            g = base + jl

            @pl.when(jnp.logical_and(g >= lo, g < hi))
            def _():
                work()

        UNR = 4 if nk > 4 else 8
        GN = gch // CH
        slots = (rows0, rows1)
        sems = (sem0, sem1)

        def group_body(gi, car):
            gbase = gi * gch
            pltpu.sync_copy(gidx_h.at[pl.ds(gbase, gch)], idxg)
            pltpu.sync_copy(seg_h.at[pl.ds(gbase, gch)],
                            cidg.at[pl.ds(0, gch)])
            hs = [None] * GN
            hs[0] = pltpu.async_copy(
                table_h.at[idxg.at[pl.ds(0, CH)]], slots[0], sems[0])
            for sub in range(GN):
                slot = sub % 2
                if sub + 1 < GN:
                    hs[sub + 1] = pltpu.async_copy(
                        table_h.at[idxg.at[pl.ds((sub + 1) * CH, CH)]],
                        slots[1 - slot], sems[1 - slot])
                hs[sub].wait()

                def blk(b, c, sub=sub, slot=slot):
                    for u in range(UNR):
                        jr = b * UNR + u
                        lit_step(sub * CH + jr, jr, gbase, slots[slot])
                    return c

                lax.fori_loop(0, CH // UNR, blk, 0)
            return car

        g0 = lo // gch
        g1 = (hi + gch - 1) // gch
        lax.fori_loop(g0, g1, group_body, 0)

        finalize_and_scatter()

    return k(table, gidx, segid, bnd)


def _worker_bounds(seg_sorted, nrow):
    pos = jnp.arange(NW, dtype=jnp.int32) * (L // NW)
    heads = seg_sorted[pos]
    b = jnp.searchsorted(seg_sorted, heads, side="left").astype(jnp.int32)
    blit = jnp.concatenate([b, jnp.array([L], jnp.int32)])
    capped = jnp.minimum(blit[:NW], L - 1)
    brow_head = jnp.where(blit[:NW] < L, seg_sorted[capped],
                          jnp.int32(nrow))
    brow = jnp.concatenate([brow_head, jnp.array([nrow], jnp.int32)])
    brow = brow.at[0].set(0)
    return jnp.concatenate([jnp.pad(blit, (0, 15)),
                            jnp.pad(brow, (0, BND - 48 - 33))])


def kernel(clause_var_idx, clause_sign, clause_ids, params):
    sign = clause_sign.astype(jnp.int32)
    inv = 1 - sign
    gidx2 = clause_var_idx + V * inv
    dst4 = 2 * clause_var_idx + inv
    src4 = 2 * clause_ids + inv
    sd, ss = lax.sort([dst4, src4], num_keys=1)

    bnd2 = _worker_bounds(clause_ids, C_PAD)
    bnd4 = _worker_bounds(sd, VP2)

    pad = LP - L
    gidx2p = jnp.pad(gidx2, (0, pad))
    cidsp = jnp.pad(clause_ids, (0, pad))
    ssp = jnp.pad(ss, (0, pad))
    sdp = jnp.pad(sd, (0, pad))

    variables = 0.25 * jax.random.truncated_normal(
        jax.random.key(1), -2.0, 2.0, (V, FM), dtype=jnp.float32)

    step_logits = []
    for _ in range(ROUNDS):
        q = _tc_mlp3(variables, params["variables_query"])
        qext = jnp.concatenate([q, -q], axis=0)
        closs_full = _sc_seg_reduce(True, FM, C_PAD, qext, gidx2p, cidsp,
                                    bnd2)
        lc = _tc_loss_mlps(closs_full[:C], params["query_pos_inter"],
                           params["query_neg_inter"])
        msg_full = _sc_seg_reduce(False, FM // 2, VP2,
                                  lc.reshape(2 * C, FM // 2), ssp, sdp, bnd4)
        msg = msg_full[:2 * V].reshape(V, FM)
        variables, logit = _tc_gates(variables, msg, params["forget_gate"],
                                     params["update_gate"],
                                     params["variables_output"],
                                     params["ln_gamma"], params["ln_beta"])
        step_logits.append(logit)
    return jnp.stack(step_logits, axis=0)


# speedup vs baseline: 6.5745x; 1.0668x over previous
import functools

import jax
import jax.numpy as jnp
from jax import lax
from jax.experimental import pallas as pl
from jax.experimental.pallas import tpu as pltpu
from jax.experimental.pallas import tpu_sc as plsc

V = 50000
C = 200000
L = 600000
FM = 128
ROUNDS = 4

NW = 32
CH = 128
ST = 256
GCH = 1024
BND = 96
LP = ((L + CH - 1) // CH) * CH
C_PAD = C + 16
VP2 = 2 * V + 16


def _bcast(b, n):
    return jnp.broadcast_to(b, (n,))


def _tc_mlp3(x, layers):
    n = x.shape[0]
    br = 400
    grid = n // br
    (w1, b1), (w2, b2), (w3, b3) = layers
    bst = jnp.zeros((8, FM), jnp.float32)
    bst = bst.at[0].set(b1).at[1].set(b2).at[2].set(b3)

    def body(x_ref, w1r, w2r, w3r, br_, o_ref):
        h = x_ref[...] @ w1r[...] + br_[0:1, :]
        h = jnp.where(h >= 0, h, 0.2 * h)
        h = h @ w2r[...] + br_[1:2, :]
        h = jnp.where(h >= 0, h, 0.2 * h)
        o_ref[...] = h @ w3r[...] + br_[2:3, :]

    wspec = pl.BlockSpec((FM, FM), lambda i: (0, 0))
    return pl.pallas_call(
        body,
        grid=(grid,),
        in_specs=[pl.BlockSpec((br, FM), lambda i: (i, 0)),
                  wspec, wspec, wspec,
                  pl.BlockSpec((8, FM), lambda i: (0, 0))],
        out_specs=pl.BlockSpec((br, FM), lambda i: (i, 0)),
        out_shape=jax.ShapeDtypeStruct((n, FM), jnp.float32),
    )(x, w1, w2, w3, bst)


def _tc_loss_mlps(closs, pos_layers, neg_layers):
    br = 1600
    grid = C // br
    (pw1, pb1), (pw2, pb2), (pw3, pb3) = pos_layers
    (nw1, nb1), (nw2, nb2), (nw3, nb3) = neg_layers
    bst = jnp.zeros((8, FM), jnp.float32)
    bst = bst.at[0].set(pb1).at[1].set(pb2).at[2].set(nb1).at[3].set(nb2)
    bst = bst.at[4].set(jnp.concatenate([pb3, nb3]))

    def body(x_ref, pw1r, pw2r, pw3r, nw1r, nw2r, nw3r, br_, o_ref):
        x = x_ref[...]
        h = x @ pw1r[...] + br_[0:1, :]
        h = jnp.where(h >= 0, h, 0.2 * h)
        h = h @ pw2r[...] + br_[1:2, :]
        h = jnp.where(h >= 0, h, 0.2 * h)
        hp = h @ pw3r[...]
        g = x @ nw1r[...] + br_[2:3, :]
        g = jnp.where(g >= 0, g, 0.2 * g)
        g = g @ nw2r[...] + br_[3:4, :]
        g = jnp.where(g >= 0, g, 0.2 * g)
        hn = g @ nw3r[...]
        o_ref[...] = jnp.concatenate([hp, hn], axis=1) + br_[4:5, :]

    wspec = pl.BlockSpec((FM, FM), lambda i: (0, 0))
    hspec = pl.BlockSpec((FM, FM // 2), lambda i: (0, 0))
    return pl.pallas_call(
        body,
        grid=(grid,),
        in_specs=[pl.BlockSpec((br, FM), lambda i: (i, 0)),
                  wspec, wspec, hspec, wspec, wspec, hspec,
                  pl.BlockSpec((8, FM), lambda i: (0, 0))],
        out_specs=pl.BlockSpec((br, FM), lambda i: (i, 0)),
        out_shape=jax.ShapeDtypeStruct((C, FM), jnp.float32),
    )(closs, pw1, pw2, pw3, nw1, nw2, nw3, bst)


def _tc_gates(variables, msg, fg, ug, og, gamma, beta):
    br = 400
    grid = V // br
    (fw1, fb1), (fw2, fb2), (fw3, fb3) = fg
    (uw1, ub1), (uw2, ub2), (uw3, ub3) = ug
    (ow1, ob1), (ow2, ob2), (ow3, ob3) = og
    bst = jnp.zeros((8, FM), jnp.float32)
    bst = (bst.at[0].set(fb1).at[1].set(fb2).at[2].set(fb3)
              .at[3].set(ub1).at[4].set(ub2).at[5].set(ub3)
              .at[6].set(ob1).at[7].set(ob2))
    bst2 = jnp.zeros((8, FM), jnp.float32)
    bst2 = (bst2.at[0].set(gamma).at[1].set(beta)
                .at[2].set(ow3[:, 0]).at[3].set(_bcast(ob3[0], FM)))

    def body(v_ref, m_ref, fw1v, fw1m, fw2r, fw3r,
             uw1v, uw1m, uw2r, uw3r, ow1r, ow2r, br_, br2_, nv_ref, lg_ref):
        v = v_ref[...]
        m = m_ref[...]
        h = v @ fw1v[...] + m @ fw1m[...] + br_[0:1, :]
        h = jnp.where(h >= 0, h, 0.2 * h)
        h = h @ fw2r[...] + br_[1:2, :]
        h = jnp.where(h >= 0, h, 0.2 * h)
        f = jax.nn.sigmoid(h @ fw3r[...] + br_[2:3, :])
        h = v @ uw1v[...] + m @ uw1m[...] + br_[3:4, :]
        h = jnp.where(h >= 0, h, 0.2 * h)
        h = h @ uw2r[...] + br_[4:5, :]
        h = jnp.where(h >= 0, h, 0.2 * h)
        u = h @ uw3r[...] + br_[5:6, :]
        nv = (1.0 - f) * v + f * u
        mu = jnp.mean(nv, axis=1, keepdims=True)
        d = nv - mu
        var = jnp.mean(d * d, axis=1, keepdims=True)
        ln = d * lax.rsqrt(var + 1e-3) * br2_[0:1, :] + br2_[1:2, :]
        nv_ref[...] = ln
        h = ln @ ow1r[...] + br_[6:7, :]
        h = jnp.where(h >= 0, h, 0.2 * h)
        h = h @ ow2r[...] + br_[7:8, :]
        h = jnp.where(h >= 0, h, 0.2 * h)
        lg_ref[...] = (jnp.sum(h * br2_[2:3, :], axis=1, keepdims=True)
                       + br2_[3:4, 0:1])

    wspec = pl.BlockSpec((FM, FM), lambda i: (0, 0))
    bspec = pl.BlockSpec((8, FM), lambda i: (0, 0))
    return pl.pallas_call(
        body,
        grid=(grid,),
        in_specs=[pl.BlockSpec((br, FM), lambda i: (i, 0)),
                  pl.BlockSpec((br, FM), lambda i: (i, 0)),
                  wspec, wspec, wspec, wspec,
                  wspec, wspec, wspec, wspec,
                  wspec, wspec, bspec, bspec],
        out_specs=[pl.BlockSpec((br, FM), lambda i: (i, 0)),
                   pl.BlockSpec((br, 1), lambda i: (i, 0))],
        out_shape=[jax.ShapeDtypeStruct((V, FM), jnp.float32),
                   jax.ShapeDtypeStruct((V, 1), jnp.float32)],
    )(variables, msg, fw1[:FM], fw1[FM:], fw2, fw3,
      uw1[:FM], uw1[FM:], uw2, uw3, ow1, ow2, bst, bst2)


def _sc_seg_reduce(is_prod, ncol, nrow, table, gidx, segid, bnd):
    nk = ncol // 16
    ident = 1.0 if is_prod else 0.0
    gch = GCH // 2 if nk > 4 else GCH
    mesh = plsc.VectorSubcoreMesh(core_axis_name="c", subcore_axis_name="s")

    @functools.partial(
        pl.kernel,
        out_type=jax.ShapeDtypeStruct((nrow, ncol), jnp.float32),
        mesh=mesh,
        compiler_params=pltpu.CompilerParams(use_tc_tiling_on_sc=False),
        scratch_types=[
            pltpu.VMEM((gch,), jnp.int32),
            pltpu.VMEM((gch + 16,), jnp.int32),
            pltpu.VMEM((BND,), jnp.int32),
            pltpu.VMEM((CH, ncol), jnp.float32),
            pltpu.VMEM((CH, ncol), jnp.float32),
            pltpu.VMEM((ST, ncol), jnp.float32),
            pltpu.VMEM((ST,), jnp.int32),
            pltpu.SMEM((8,), jnp.int32),
            pltpu.SemaphoreType.DMA,
            pltpu.SemaphoreType.DMA,
        ],
    )
    def k(table_h, gidx_h, seg_h, bnd_h, out_h,
          idxg, cidg, bnds, rows0, rows1, stage, sidx, sm, sem0, sem1):
        w = lax.axis_index("c") * 16 + lax.axis_index("s")
        pltpu.sync_copy(bnd_h, bnds)
        lo = bnds[pl.ds(w, 16)][0]
        hi = bnds[pl.ds(w + 1, 16)][0]
        rlo = bnds[pl.ds(w + 48, 16)][0]
        rhi = bnds[pl.ds(w + 49, 16)][0]

        ident_v = jnp.full((16,), ident, jnp.float32)
        dump_v = jnp.full((16,), nrow - 1, jnp.int32)

        @pl.loop(0, ST)
        def _(r):
            for k2 in range(nk):
                stage[r, pl.ds(16 * k2, 16)] = ident_v

        for k2 in range(ST // 16):
            sidx[pl.ds(16 * k2, 16)] = dump_v
        sm[1] = jnp.int32(0)

        def fill_body(i, rr):
            pltpu.sync_copy(stage, out_h.at[pl.ds(rr, ST)])
            return rr + ST

        rr = lax.fori_loop(0, (rhi - rlo) // ST, fill_body, rlo)
        for s in (128, 64, 32, 16, 8, 4, 2, 1):
            do = rr + s <= rhi

            @pl.when(do)
            def _(s=s, rr=rr):
                pltpu.sync_copy(stage.at[pl.ds(0, s)], out_h.at[pl.ds(rr, s)])

            rr = jnp.where(do, rr + s, rr)

        sm[0] = jnp.int32(-1)

        def finalize_and_scatter():
            if is_prod:
                @pl.loop(0, sm[1])
                def _(r):
                    for k2 in range(nk):
                        stage[r, pl.ds(16 * k2, 16)] = (
                            1.0 / stage[r, pl.ds(16 * k2, 16)])
            pltpu.sync_copy(stage, out_h.at[sidx])

        def lit_step(jl, jr, base, rows):
            cid = cidg[pl.ds(jl, 16)][0]

            def boundary():
                @pl.when(sm[1] >= ST - 16)
                def _():
                    finalize_and_scatter()
                    sm[1] = jnp.int32(0)

                s = sm[1]
                for k2 in range(nk):
                    stage[s, pl.ds(16 * k2, 16)] = ident_v
                cid_v = jnp.full((16,), cid, jnp.int32)
                l0 = lax.iota(jnp.int32, 16) == 0
                sidx[pl.ds(s, 16)] = jnp.where(l0, cid_v, dump_v)
                sm[1] = s + 1
                sm[0] = cid

            def work():
                @pl.when(cid != sm[0])
                def _():
                    boundary()

                s1 = sm[1] - 1
                for k2 in range(nk):
                    x = rows[jr, pl.ds(16 * k2, 16)]
                    if is_prod:
                        a = stage[s1, pl.ds(16 * k2, 16)]
                        stage[s1, pl.ds(16 * k2, 16)] = (
                            a * (1.0 + jnp.exp(x)))
                    else:
                        plsc.addupdate(stage.at[s1, pl.ds(16 * k2, 16)], x)

            g = base + jl

            @pl.when(jnp.logical_and(g >= lo, g < hi))
            def _():
                work()

        UNR = 4 if nk > 4 else 8
        GN = gch // CH
        slots = (rows0, rows1)
        sems = (sem0, sem1)

        def group_body(gi, car):
            gbase = gi * gch
            pltpu.sync_copy(gidx_h.at[pl.ds(gbase, gch)], idxg)
            pltpu.sync_copy(seg_h.at[pl.ds(gbase, gch)],
                            cidg.at[pl.ds(0, gch)])
            hs = [None] * GN
            hs[0] = pltpu.async_copy(
                table_h.at[idxg.at[pl.ds(0, CH)]], slots[0], sems[0])
            for sub in range(GN):
                slot = sub % 2
                if sub + 1 < GN:
                    hs[sub + 1] = pltpu.async_copy(
                        table_h.at[idxg.at[pl.ds((sub + 1) * CH, CH)]],
                        slots[1 - slot], sems[1 - slot])
                hs[sub].wait()

                def blk(b, c, sub=sub, slot=slot):
                    for u in range(UNR):
                        jr = b * UNR + u
                        lit_step(sub * CH + jr, jr, gbase, slots[slot])
                    return c

                lax.fori_loop(0, CH // UNR, blk, 0)
            return car

        g0 = lo // gch
        g1 = (hi + gch - 1) // gch
        lax.fori_loop(g0, g1, group_body, 0)

        finalize_and_scatter()

    return k(table, gidx, segid, bnd)


def _worker_bounds(seg_sorted, nrow):
    pos = jnp.arange(NW, dtype=jnp.int32) * (L // NW)
    heads = seg_sorted[pos]
    b = jnp.searchsorted(seg_sorted, heads, side="left").astype(jnp.int32)
    blit = jnp.concatenate([b, jnp.array([L], jnp.int32)])
    capped = jnp.minimum(blit[:NW], L - 1)
    brow_head = jnp.where(blit[:NW] < L, seg_sorted[capped],
                          jnp.int32(nrow))
    brow = jnp.concatenate([brow_head, jnp.array([nrow], jnp.int32)])
    brow = brow.at[0].set(0)
    return jnp.concatenate([jnp.pad(blit, (0, 15)),
                            jnp.pad(brow, (0, BND - 48 - 33))])


def kernel(clause_var_idx, clause_sign, clause_ids, params):
    sign = clause_sign.astype(jnp.int32)
    inv = 1 - sign
    gidx2 = clause_var_idx + V * inv
    dst4 = 2 * clause_var_idx + inv
    src4 = 2 * clause_ids + inv
    sd, ss = lax.sort([dst4, src4], num_keys=1)

    bnd2 = _worker_bounds(clause_ids, C_PAD)
    bnd4 = _worker_bounds(sd, VP2)

    pad = LP - L
    gidx2p = jnp.pad(gidx2, (0, pad))
    cidsp = jnp.pad(clause_ids, (0, pad))
    ssp = jnp.pad(ss, (0, pad))
    sdp = jnp.pad(sd, (0, pad))

    variables = 0.25 * jax.random.truncated_normal(
        jax.random.key(1), -2.0, 2.0, (V, FM), dtype=jnp.float32)

    step_logits = []
    for _ in range(ROUNDS):
        q = _tc_mlp3(variables, params["variables_query"])
        qext = jnp.concatenate([q, -q], axis=0)
        closs_full = _sc_seg_reduce(True, FM, C_PAD, qext, gidx2p, cidsp,
                                    bnd2)
        lc = _tc_loss_mlps(closs_full, params["query_pos_inter"],
                           params["query_neg_inter"])
        msg_full = _sc_seg_reduce(False, FM // 2, VP2,
                                  lc.reshape(2 * C, FM // 2), ssp, sdp, bnd4)
        msg = msg_full.reshape(VP2 // 2, FM)
        variables, logit = _tc_gates(variables, msg, params["forget_gate"],
                                     params["update_gate"],
                                     params["variables_output"],
                                     params["ln_gamma"], params["ln_beta"])
        step_logits.append(logit)
    return jnp.stack(step_logits, axis=0)
